# baseline (speedup 1.0000x reference)
import functools

import jax
import jax.numpy as jnp
from jax import lax
from jax.experimental import pallas as pl
from jax.experimental.pallas import tpu as pltpu
from jax.experimental.pallas import tpu_sc as plsc

BATCH = 4096
MAXLEN = 200
EMBED = 64
LANES = 16

_info = plsc.get_sparse_core_info()
NC, NS = _info.num_cores, _info.num_subcores
NW = NC * NS
ROWS = BATCH * MAXLEN
ROWS_W = ROWS // NW
RPC = 128
NCHUNK = ROWS_W // RPC
VPR = EMBED // LANES


def _body(x_hbm, pos_hbm, tok_hbm, out_hbm, idx_v, pos_v, rows0, rows1, g0, g1):
    w = lax.axis_index("s") * NC + lax.axis_index("c")
    base_row = w * ROWS_W
    pltpu.sync_copy(x_hbm.at[w], idx_v)
    pltpu.sync_copy(pos_hbm, pos_v)
    rows = (rows0, rows1)
    sems = (g0, g1)

    def gather_start(c, b):
        pltpu.make_async_copy(tok_hbm.at[idx_v.at[c]], rows[b], sems[b]).start()

    def gather_wait(b):
        pltpu.make_async_copy(tok_hbm.at[idx_v.at[0]], rows[b], sems[b]).wait()

    def add_pos(c, b):
        pr0 = lax.rem(c * RPC, MAXLEN)
        rbuf = rows[b]

        def row(r, carry):
            pr = pr0 + r
            for k in range(VPR):
                sl = pl.ds(k * LANES, LANES)
                rbuf[r, sl] = rbuf[r, sl] + pos_v[pr, sl]
            return carry

        lax.fori_loop(0, RPC, row, 0, unroll=2)

    def store(c, b):
        pltpu.sync_copy(rows[b], out_hbm.at[pl.ds(base_row + c * RPC, RPC)])

    gather_start(0, 0)

    def outer(i, carry):
        c0 = i * 2
        gather_start(c0 + 1, 1)
        gather_wait(0)
        add_pos(c0, 0)
        store(c0, 0)

        @pl.when(c0 + 2 < NCHUNK)
        def _():
            gather_start(c0 + 2, 0)

        gather_wait(1)
        add_pos(c0 + 1, 1)
        store(c0 + 1, 1)
        return carry

    lax.fori_loop(0, NCHUNK // 2, outer, 0)


@jax.jit
def kernel(x, token_table, pos_table):
    B, T = x.shape
    V, D = token_table.shape
    assert (B, T, D) == (BATCH, MAXLEN, EMBED)
    x32 = x.astype(jnp.int32).reshape(NW, NCHUNK, RPC)
    pos2 = jnp.concatenate([pos_table, pos_table], axis=0)

    run = pl.kernel(
        _body,
        out_type=jax.ShapeDtypeStruct((ROWS, D), jnp.float32),
        mesh=plsc.VectorSubcoreMesh(core_axis_name="c", subcore_axis_name="s"),
        compiler_params=pltpu.CompilerParams(use_tc_tiling_on_sc=False),
        scratch_types=[
            pltpu.VMEM((NCHUNK, RPC), jnp.int32),
            pltpu.VMEM((2 * MAXLEN, EMBED), jnp.float32),
            pltpu.VMEM((RPC, EMBED), jnp.float32),
            pltpu.VMEM((RPC, EMBED), jnp.float32),
            pltpu.SemaphoreType.DMA,
            pltpu.SemaphoreType.DMA,
        ],
    )
    out = run(x32, pos2, token_table)
    return out.reshape(B, T, D)


# speedup vs baseline: 1.2790x; 1.2790x over previous
import functools

import jax
import jax.numpy as jnp
from jax import lax
from jax.experimental import pallas as pl
from jax.experimental.pallas import tpu as pltpu
from jax.experimental.pallas import tpu_sc as plsc

BATCH = 4096
MAXLEN = 200
EMBED = 64
LANES = 16

_info = plsc.get_sparse_core_info()
NC, NS = _info.num_cores, _info.num_subcores
NW = NC * NS
BPW = BATCH // NW
VPR = EMBED // LANES


def _body(x_hbm, pos_hbm, tok_hbm, out_hbm, idx_v, pos_v, rows0, rows1, g0, g1):
    w = lax.axis_index("s") * NC + lax.axis_index("c")
    pltpu.sync_copy(x_hbm.at[w], idx_v)
    pltpu.sync_copy(pos_hbm, pos_v)
    rows = (rows0, rows1)
    sems = (g0, g1)

    def gather_start(t, b):
        pltpu.make_async_copy(tok_hbm.at[idx_v.at[t]], rows[b], sems[b]).start()

    def gather_wait(b):
        pltpu.make_async_copy(tok_hbm.at[idx_v.at[0]], rows[b], sems[b]).wait()

    def add_pos(t, b):
        rbuf = rows[b]
        pv = [pos_v[t, pl.ds(k * LANES, LANES)] for k in range(VPR)]

        def row(r, carry):
            for k in range(VPR):
                sl = pl.ds(k * LANES, LANES)
                rbuf[r, sl] = rbuf[r, sl] + pv[k]
            return carry

        lax.fori_loop(0, BPW, row, 0, unroll=4)

    def store(t, b):
        pltpu.sync_copy(rows[b], out_hbm.at[w, :, t])

    gather_start(0, 0)

    def outer(i, carry):
        t0 = i * 2
        gather_start(t0 + 1, 1)
        gather_wait(0)
        add_pos(t0, 0)
        store(t0, 0)

        @pl.when(t0 + 2 < MAXLEN)
        def _():
            gather_start(t0 + 2, 0)

        gather_wait(1)
        add_pos(t0 + 1, 1)
        store(t0 + 1, 1)
        return carry

    lax.fori_loop(0, MAXLEN // 2, outer, 0)


@jax.jit
def kernel(x, token_table, pos_table):
    B, T = x.shape
    V, D = token_table.shape
    assert (B, T, D) == (BATCH, MAXLEN, EMBED)
    x32 = x.astype(jnp.int32).reshape(NW, BPW, T).transpose(0, 2, 1)

    run = pl.kernel(
        _body,
        out_type=jax.ShapeDtypeStruct((NW, BPW, T, D), jnp.float32),
        mesh=plsc.VectorSubcoreMesh(core_axis_name="c", subcore_axis_name="s"),
        compiler_params=pltpu.CompilerParams(use_tc_tiling_on_sc=False),
        scratch_types=[
            pltpu.VMEM((T, BPW), jnp.int32),
            pltpu.VMEM((T, EMBED), jnp.float32),
            pltpu.VMEM((BPW, EMBED), jnp.float32),
            pltpu.VMEM((BPW, EMBED), jnp.float32),
            pltpu.SemaphoreType.DMA,
            pltpu.SemaphoreType.DMA,
        ],
    )
    out = run(x32, pos_table, token_table)
    return out.reshape(B, T, D)
